# speedup vs baseline: 4.5529x; 4.5529x over previous
import jax
import jax.numpy as jnp
from jax import lax
from jax.experimental import pallas as pl
from jax.experimental.pallas import tpu as pltpu
from jax.experimental.pallas import tpu_sc as plsc

N = 1024
E = 16384
FDIM = 128
JACOBI = 3


_NC, _NS = 2, 16
_NW = _NC * _NS
_EPW = E // _NW
_CHUNKS = _EPW // 128
_WORDS_PER_TILE = (N * N) // _NS
_ZBUF = 8192


def _sc_scatter_kernel(rows_hbm, cols_hbm, vals_hbm, a0_hbm, a1_hbm,
                      rows_v, cols_v, vals_v, idx_v, zb_v, acc_sh):
    cid = lax.axis_index("c")
    sid = lax.axis_index("s")
    wid = cid * _NS + sid
    base = wid * _EPW

    pltpu.sync_copy(rows_hbm.at[pl.ds(base, _EPW)], rows_v)
    pltpu.sync_copy(cols_hbm.at[pl.ds(base, _EPW)], cols_v)
    pltpu.sync_copy(vals_hbm.at[pl.ds(base, _EPW)], vals_v)

    for j in range(_CHUNKS):
        for k in range(128 // 16):
            e = j * 128 + k * 16
            r = rows_v[pl.ds(e, 16)]
            c = cols_v[pl.ds(e, 16)]
            idx_v[j, pl.ds(k * 16, 16)] = r * N + c

    zeros16 = jnp.zeros((16,), jnp.float32)

    def _zfill(i, carry):
        zb_v[pl.ds(i * 16, 16)] = zeros16
        return carry

    lax.fori_loop(0, _ZBUF // 16, _zfill, 0)
    for j in range(_WORDS_PER_TILE // _ZBUF):
        pltpu.sync_copy(
            zb_v, acc_sh.at[pl.ds(sid * _WORDS_PER_TILE + j * _ZBUF, _ZBUF)])

    plsc.subcore_barrier()

    for j in range(_CHUNKS):
        pltpu.sync_copy(vals_v.at[pl.ds(j * 128, 128)],
                        acc_sh.at[idx_v.at[j]], add=True)

    plsc.subcore_barrier()

    @pl.when(cid == 0)
    def _():
        pltpu.sync_copy(
            acc_sh.at[pl.ds(sid * _WORDS_PER_TILE, _WORDS_PER_TILE)],
            a0_hbm.at[pl.ds(sid * _WORDS_PER_TILE, _WORDS_PER_TILE)])

    @pl.when(cid == 1)
    def _():
        pltpu.sync_copy(
            acc_sh.at[pl.ds(sid * _WORDS_PER_TILE, _WORDS_PER_TILE)],
            a1_hbm.at[pl.ds(sid * _WORDS_PER_TILE, _WORDS_PER_TILE)])


def _build_adjacency(rows, cols, vals):
    mesh = plsc.VectorSubcoreMesh(core_axis_name="c", subcore_axis_name="s")
    flat = jax.ShapeDtypeStruct((N * N,), jnp.float32)
    fn = pl.kernel(
        _sc_scatter_kernel,
        mesh=mesh,
        out_type=[flat, flat],
        scratch_types=[
            pltpu.VMEM((_EPW,), jnp.int32),
            pltpu.VMEM((_EPW,), jnp.int32),
            pltpu.VMEM((_EPW,), jnp.float32),
            pltpu.VMEM((_CHUNKS, 128), jnp.int32),
            pltpu.VMEM((_ZBUF,), jnp.float32),
            pltpu.VMEM_SHARED((N * N,), jnp.float32),
        ],
    )
    return fn(rows, cols, vals)


def _tc_kernel(h_ref, a0_ref, a1_ref, x_ref, w0_ref, wr_ref, wi_ref, o_ref):
    h = h_ref[0]
    A = a0_ref[:] + a1_ref[:]
    x = x_ref[:]

    dr = h * (1.0 - jnp.sum(A, axis=1))
    denom = dr * dr + 1.0
    ar = (dr / denom)[:, None]
    ai = (-1.0 / denom)[:, None]

    def S(u):
        return h * (u - jnp.dot(A, u, preferred_element_type=jnp.float32))

    def m_apply(vr, vi):
        xr = S(vr) + vi
        xi = S(vi) - vr
        yr, yi = xr, xi
        for _ in range(JACOBI):
            tr = S(yr)
            ti = S(yi)
            rr = xr - (tr - yi)
            ri = xi - (ti + yr)
            yr = yr + ar * rr - ai * ri
            yi = yi + ar * ri + ai * rr
        return yr, yi

    def proj(w):
        return jnp.dot(x, w, preferred_element_type=jnp.float32)

    zr, zi = m_apply(proj(wr_ref[1]), proj(wi_ref[1]))
    yr, _ = m_apply(proj(wr_ref[0]) + zr, proj(wi_ref[0]) + zi)
    o_ref[:] = jnp.maximum(proj(w0_ref[:]) + 2.0 * yr, 0.0)


def _dense_cayley(h, a0, a1, x, w0, wr, wi):
    vspec = pl.BlockSpec(memory_space=pltpu.VMEM)
    return pl.pallas_call(
        _tc_kernel,
        out_shape=jax.ShapeDtypeStruct((N, FDIM), jnp.float32),
        in_specs=[pl.BlockSpec(memory_space=pltpu.SMEM),
                  vspec, vspec, vspec, vspec, vspec, vspec],
        out_specs=vspec,
    )(h, a0, a1, x, w0, wr, wi)


def kernel(x, adj_indices, adj_values, h, W0, W_real, W_imag):
    rows = adj_indices[0].astype(jnp.int32)
    cols = adj_indices[1].astype(jnp.int32)
    vals = adj_values.astype(jnp.float32)
    a0_flat, a1_flat = _build_adjacency(rows, cols, vals)
    a0 = a0_flat.reshape(N, N)
    a1 = a1_flat.reshape(N, N)
    h_arr = jnp.asarray(h, jnp.float32).reshape(1)
    return _dense_cayley(h_arr, a0, a1, x, W0, W_real, W_imag)


# speedup vs baseline: 5.1581x; 1.1329x over previous
import jax
import jax.numpy as jnp
from jax import lax
from jax.experimental import pallas as pl
from jax.experimental.pallas import tpu as pltpu
from jax.experimental.pallas import tpu_sc as plsc

N = 1024
E = 16384
FDIM = 128
JACOBI = 3

_NC, _NS = 2, 16
_EPT = E // _NS
_CHUNKS = _EPT // 128
_HALF = (N * N) // _NC
_WPT = _HALF // _NS
_ZBUF = 8192
_DUMP = _HALF


def _sc_scatter_kernel(rows_hbm, cols_hbm, vals_hbm, a_hbm,
                       rows_v, cols_v, vals_v, idx_v, zb_v, acc_sh):
    cid = lax.axis_index("c")
    sid = lax.axis_index("s")
    base = sid * _EPT

    pltpu.sync_copy(rows_hbm.at[pl.ds(base, _EPT)], rows_v)
    pltpu.sync_copy(cols_hbm.at[pl.ds(base, _EPT)], cols_v)
    pltpu.sync_copy(vals_hbm.at[pl.ds(base, _EPT)], vals_v)

    row_lo = cid * (N // _NC)
    for j in range(_CHUNKS):
        for k in range(128 // 16):
            e = j * 128 + k * 16
            r = rows_v[pl.ds(e, 16)] - row_lo
            c = cols_v[pl.ds(e, 16)]
            flat = r * N + c
            ok = (r >= 0) & (r < (N // _NC))
            idx_v[j, pl.ds(k * 16, 16)] = jnp.where(ok, flat, _DUMP + sid * 16)

    zeros16 = jnp.zeros((16,), jnp.float32)

    def _zfill(i, carry):
        zb_v[pl.ds(i * 16, 16)] = zeros16
        return carry

    lax.fori_loop(0, _ZBUF // 16, _zfill, 0)
    for j in range(_WPT // _ZBUF):
        pltpu.sync_copy(zb_v, acc_sh.at[pl.ds(sid * _WPT + j * _ZBUF, _ZBUF)])

    plsc.subcore_barrier()

    for j in range(_CHUNKS):
        pltpu.sync_copy(vals_v.at[pl.ds(j * 128, 128)],
                        acc_sh.at[idx_v.at[j]], add=True)

    plsc.subcore_barrier()

    pltpu.sync_copy(acc_sh.at[pl.ds(sid * _WPT, _WPT)],
                    a_hbm.at[pl.ds(cid * _HALF + sid * _WPT, _WPT)])


def _build_adjacency(rows, cols, vals):
    mesh = plsc.VectorSubcoreMesh(core_axis_name="c", subcore_axis_name="s")
    fn = pl.kernel(
        _sc_scatter_kernel,
        mesh=mesh,
        out_type=jax.ShapeDtypeStruct((N * N,), jnp.float32),
        scratch_types=[
            pltpu.VMEM((_EPT,), jnp.int32),
            pltpu.VMEM((_EPT,), jnp.int32),
            pltpu.VMEM((_EPT,), jnp.float32),
            pltpu.VMEM((_CHUNKS, 128), jnp.int32),
            pltpu.VMEM((_ZBUF,), jnp.float32),
            pltpu.VMEM_SHARED((_HALF + 16 * _NS,), jnp.float32),
        ],
    )
    return fn(rows, cols, vals)


def _tc_kernel(h_ref, a_ref, x_ref, w0_ref, wr_ref, wi_ref, o_ref):
    h = h_ref[0]
    A = a_ref[:]
    x = x_ref[:]

    dr = h * (1.0 - jnp.sum(A, axis=1))
    denom = dr * dr + 1.0
    ar = (dr / denom)[:, None]
    ai = (-1.0 / denom)[:, None]

    def S(u):
        return h * (u - jnp.dot(A, u, preferred_element_type=jnp.float32))

    def m_apply(vr, vi):
        xr = S(vr) + vi
        xi = S(vi) - vr
        yr, yi = xr, xi
        for _ in range(JACOBI):
            tr = S(yr)
            ti = S(yi)
            rr = xr - (tr - yi)
            ri = xi - (ti + yr)
            yr = yr + ar * rr - ai * ri
            yi = yi + ar * ri + ai * rr
        return yr, yi

    def proj(w):
        return jnp.dot(x, w, preferred_element_type=jnp.float32)

    zr, zi = m_apply(proj(wr_ref[1]), proj(wi_ref[1]))
    yr, _ = m_apply(proj(wr_ref[0]) + zr, proj(wi_ref[0]) + zi)
    o_ref[:] = jnp.maximum(proj(w0_ref[:]) + 2.0 * yr, 0.0)


def _dense_cayley(h, a, x, w0, wr, wi):
    vspec = pl.BlockSpec(memory_space=pltpu.VMEM)
    return pl.pallas_call(
        _tc_kernel,
        out_shape=jax.ShapeDtypeStruct((N, FDIM), jnp.float32),
        in_specs=[pl.BlockSpec(memory_space=pltpu.SMEM),
                  vspec, vspec, vspec, vspec, vspec],
        out_specs=vspec,
    )(h, a, x, w0, wr, wi)


def kernel(x, adj_indices, adj_values, h, W0, W_real, W_imag):
    rows = adj_indices[0].astype(jnp.int32)
    cols = adj_indices[1].astype(jnp.int32)
    vals = adj_values.astype(jnp.float32)
    a = _build_adjacency(rows, cols, vals).reshape(N, N)
    h_arr = jnp.asarray(h, jnp.float32).reshape(1)
    return _dense_cayley(h_arr, a, x, W0, W_real, W_imag)


# speedup vs baseline: 5.8300x; 1.1303x over previous
import jax
import jax.numpy as jnp
from jax import lax
from jax.experimental import pallas as pl
from jax.experimental.pallas import tpu as pltpu
from jax.experimental.pallas import tpu_sc as plsc

N = 1024
E = 16384
FDIM = 128
JACOBI = 3

_NC, _NS = 2, 16
_EPT = E // _NS
_CHUNKS = _EPT // 128
_HALF = (N * N) // _NC
_WPT = _HALF // _NS
_ZBUF = 8192
_DUMP = _HALF


def _sc_scatter_kernel(adjf_hbm, vals_hbm, a_hbm,
                       rows_v, cols_v, vals_v, idx_v, zb_v, acc_sh):
    cid = lax.axis_index("c")
    sid = lax.axis_index("s")
    base = sid * _EPT

    pltpu.sync_copy(adjf_hbm.at[pl.ds(base, _EPT)], rows_v)
    pltpu.sync_copy(adjf_hbm.at[pl.ds(E + base, _EPT)], cols_v)
    pltpu.sync_copy(vals_hbm.at[pl.ds(base, _EPT)], vals_v)

    row_lo = cid * (N // _NC)
    for j in range(_CHUNKS):
        for k in range(128 // 16):
            e = j * 128 + k * 16
            r = rows_v[pl.ds(e, 16)] - row_lo
            c = cols_v[pl.ds(e, 16)]
            flat = r * N + c
            ok = (r >= 0) & (r < (N // _NC))
            idx_v[j, pl.ds(k * 16, 16)] = jnp.where(ok, flat, _DUMP + sid * 16)

    zeros16 = jnp.zeros((16,), jnp.float32)

    def _zfill(i, carry):
        zb_v[pl.ds(i * 16, 16)] = zeros16
        return carry

    lax.fori_loop(0, _ZBUF // 16, _zfill, 0)
    for j in range(_WPT // _ZBUF):
        pltpu.sync_copy(zb_v, acc_sh.at[pl.ds(sid * _WPT + j * _ZBUF, _ZBUF)])

    plsc.subcore_barrier()

    for j in range(_CHUNKS):
        pltpu.sync_copy(vals_v.at[pl.ds(j * 128, 128)],
                        acc_sh.at[idx_v.at[j]], add=True)

    plsc.subcore_barrier()

    pltpu.sync_copy(acc_sh.at[pl.ds(sid * _WPT, _WPT)],
                    a_hbm.at[pl.ds(cid * _HALF + sid * _WPT, _WPT)])


def _build_adjacency(adj_flat, vals):
    mesh = plsc.VectorSubcoreMesh(core_axis_name="c", subcore_axis_name="s")
    fn = pl.kernel(
        _sc_scatter_kernel,
        mesh=mesh,
        out_type=jax.ShapeDtypeStruct((N * N,), jnp.float32),
        scratch_types=[
            pltpu.VMEM((_EPT,), jnp.int32),
            pltpu.VMEM((_EPT,), jnp.int32),
            pltpu.VMEM((_EPT,), jnp.float32),
            pltpu.VMEM((_CHUNKS, 128), jnp.int32),
            pltpu.VMEM((_ZBUF,), jnp.float32),
            pltpu.VMEM_SHARED((_HALF + 16 * _NS,), jnp.float32),
        ],
    )
    return fn(adj_flat, vals)


def _tc_kernel(h_ref, a_hbm, x_ref, w0_ref, wr_ref, wi_ref, o_ref,
               a_vmem, a_sem):
    h = h_ref[0]
    x = x_ref[:]

    cp = pltpu.make_async_copy(a_hbm.reshape(N, N), a_vmem, a_sem)
    cp.start()

    wcat = jnp.concatenate(
        [w0_ref[:], wr_ref[1], wi_ref[1], wr_ref[0], wi_ref[0]], axis=1)
    p = jnp.dot(x, wcat, preferred_element_type=jnp.float32)

    cp.wait()
    A = a_vmem[:]

    dr = h * (1.0 - jnp.sum(A, axis=1))
    denom = dr * dr + 1.0
    ar = (dr / denom)[:, None]
    ai = (-1.0 / denom)[:, None]

    def S(u):
        return h * (u - jnp.dot(A, u, preferred_element_type=jnp.float32))

    def m_apply(v):
        sv = S(v)
        vr, vi = v[:, :FDIM], v[:, FDIM:]
        xr = sv[:, :FDIM] + vi
        xi = sv[:, FDIM:] - vr
        yr, yi = xr, xi
        for _ in range(JACOBI):
            t = S(jnp.concatenate([yr, yi], axis=1))
            rr = xr - (t[:, :FDIM] - yi)
            ri = xi - (t[:, FDIM:] + yr)
            yr = yr + ar * rr - ai * ri
            yi = yi + ar * ri + ai * rr
        return yr, yi

    p0 = p[:, :FDIM]
    zr, zi = m_apply(p[:, FDIM:3 * FDIM])
    yr, _ = m_apply(
        jnp.concatenate([p[:, 3 * FDIM:4 * FDIM] + zr,
                         p[:, 4 * FDIM:] + zi], axis=1))
    o_ref[:] = jnp.maximum(p0 + 2.0 * yr, 0.0)


def _dense_cayley(h, a_flat, x, w0, wr, wi):
    vspec = pl.BlockSpec(memory_space=pltpu.VMEM)
    return pl.pallas_call(
        _tc_kernel,
        out_shape=jax.ShapeDtypeStruct((N, FDIM), jnp.float32),
        in_specs=[pl.BlockSpec(memory_space=pltpu.SMEM),
                  pl.BlockSpec(memory_space=pl.ANY),
                  vspec, vspec, vspec, vspec],
        out_specs=vspec,
        scratch_shapes=[pltpu.VMEM((N, N), jnp.float32),
                        pltpu.SemaphoreType.DMA],
    )(h, a_flat, x, w0, wr, wi)


def kernel(x, adj_indices, adj_values, h, W0, W_real, W_imag):
    adj_flat = adj_indices.astype(jnp.int32).reshape(2 * E)
    vals = adj_values.astype(jnp.float32)
    a_flat = _build_adjacency(adj_flat, vals).reshape(N * N // 128, 128)
    h_arr = jnp.asarray(h, jnp.float32).reshape(1)
    return _dense_cayley(h_arr, a_flat, x, W0, W_real, W_imag)


# speedup vs baseline: 6.1353x; 1.0524x over previous
import jax
import jax.numpy as jnp
from jax import lax
from jax.experimental import pallas as pl
from jax.experimental.pallas import tpu as pltpu
from jax.experimental.pallas import tpu_sc as plsc

N = 1024
E = 16384
FDIM = 128
JACOBI = 3

_NC, _NS = 2, 16
_EPT = E // _NS
_CHUNKS = _EPT // 128
_HALF = (N * N) // _NC
_WPT = _HALF // _NS
_ZBUF = 8192
_DUMP = _HALF


def _sc_scatter_kernel(adjf_hbm, vals_hbm, a_hbm,
                       rows_v, cols_v, vals_v, idx_v, zb_v, acc_sh,
                       sem_rc, sem_v, sem_z):
    cid = lax.axis_index("c")
    sid = lax.axis_index("s")
    base = sid * _EPT

    c_r = pltpu.async_copy(adjf_hbm.at[pl.ds(base, _EPT)], rows_v, sem_rc)
    c_c = pltpu.async_copy(adjf_hbm.at[pl.ds(E + base, _EPT)], cols_v, sem_rc)
    c_v = pltpu.async_copy(vals_hbm.at[pl.ds(base, _EPT)], vals_v, sem_v)

    zeros16 = jnp.zeros((16,), jnp.float32)

    def _zfill(i, carry):
        zb_v[pl.ds(i * 16, 16)] = zeros16
        return carry

    lax.fori_loop(0, _ZBUF // 16, _zfill, 0)
    zcs = [pltpu.async_copy(
        zb_v, acc_sh.at[pl.ds(sid * _WPT + j * _ZBUF, _ZBUF)], sem_z)
        for j in range(_WPT // _ZBUF)]

    c_r.wait()
    c_c.wait()
    row_lo = cid * (N // _NC)
    for j in range(_CHUNKS):
        for k in range(128 // 16):
            e = j * 128 + k * 16
            r = rows_v[pl.ds(e, 16)] - row_lo
            c = cols_v[pl.ds(e, 16)]
            flat = r * N + c
            ok = (r >= 0) & (r < (N // _NC))
            idx_v[j, pl.ds(k * 16, 16)] = jnp.where(ok, flat, _DUMP + sid * 16)

    c_v.wait()
    for z in zcs:
        z.wait()

    plsc.subcore_barrier()

    for j in range(_CHUNKS):
        pltpu.sync_copy(vals_v.at[pl.ds(j * 128, 128)],
                        acc_sh.at[idx_v.at[j]], add=True)

    plsc.subcore_barrier()

    pltpu.sync_copy(acc_sh.at[pl.ds(sid * _WPT, _WPT)],
                    a_hbm.at[pl.ds(cid * _HALF + sid * _WPT, _WPT)])


def _build_adjacency(adj_flat, vals):
    mesh = plsc.VectorSubcoreMesh(core_axis_name="c", subcore_axis_name="s")
    fn = pl.kernel(
        _sc_scatter_kernel,
        mesh=mesh,
        out_type=jax.ShapeDtypeStruct((N * N,), jnp.float32),
        scratch_types=[
            pltpu.VMEM((_EPT,), jnp.int32),
            pltpu.VMEM((_EPT,), jnp.int32),
            pltpu.VMEM((_EPT,), jnp.float32),
            pltpu.VMEM((_CHUNKS, 128), jnp.int32),
            pltpu.VMEM((_ZBUF,), jnp.float32),
            pltpu.VMEM_SHARED((_HALF + 16 * _NS,), jnp.float32),
            pltpu.SemaphoreType.DMA,
            pltpu.SemaphoreType.DMA,
            pltpu.SemaphoreType.DMA,
        ],
    )
    return fn(adj_flat, vals)


def _tc_kernel(h_ref, a_hbm, x_ref, w0_ref, wr_ref, wi_ref, o_ref,
               a_vmem, a_sem):
    h = h_ref[0]
    x = x_ref[:]

    cp = pltpu.make_async_copy(a_hbm.reshape(N, N), a_vmem, a_sem)
    cp.start()

    wcat = jnp.concatenate(
        [w0_ref[:], wr_ref[1], wi_ref[1], wr_ref[0], wi_ref[0]], axis=1)
    p = jnp.dot(x, wcat, preferred_element_type=jnp.float32)

    cp.wait()
    A = a_vmem[:]

    dr = h * (1.0 - jnp.sum(A, axis=1))
    denom = dr * dr + 1.0
    ar = (dr / denom)[:, None]
    ai = (-1.0 / denom)[:, None]

    def S(u):
        return h * (u - jnp.dot(A, u, preferred_element_type=jnp.float32))

    def m_apply(v):
        sv = S(v)
        vr, vi = v[:, :FDIM], v[:, FDIM:]
        xr = sv[:, :FDIM] + vi
        xi = sv[:, FDIM:] - vr
        yr, yi = xr, xi
        for _ in range(JACOBI):
            t = S(jnp.concatenate([yr, yi], axis=1))
            rr = xr - (t[:, :FDIM] - yi)
            ri = xi - (t[:, FDIM:] + yr)
            yr = yr + ar * rr - ai * ri
            yi = yi + ar * ri + ai * rr
        return yr, yi

    p0 = p[:, :FDIM]
    zr, zi = m_apply(p[:, FDIM:3 * FDIM])
    yr, _ = m_apply(
        jnp.concatenate([p[:, 3 * FDIM:4 * FDIM] + zr,
                         p[:, 4 * FDIM:] + zi], axis=1))
    o_ref[:] = jnp.maximum(p0 + 2.0 * yr, 0.0)


def _dense_cayley(h, a_flat, x, w0, wr, wi):
    vspec = pl.BlockSpec(memory_space=pltpu.VMEM)
    return pl.pallas_call(
        _tc_kernel,
        out_shape=jax.ShapeDtypeStruct((N, FDIM), jnp.float32),
        in_specs=[pl.BlockSpec(memory_space=pltpu.SMEM),
                  pl.BlockSpec(memory_space=pl.ANY),
                  vspec, vspec, vspec, vspec],
        out_specs=vspec,
        scratch_shapes=[pltpu.VMEM((N, N), jnp.float32),
                        pltpu.SemaphoreType.DMA],
    )(h, a_flat, x, w0, wr, wi)


def kernel(x, adj_indices, adj_values, h, W0, W_real, W_imag):
    adj_flat = adj_indices.astype(jnp.int32).reshape(2 * E)
    vals = adj_values.astype(jnp.float32)
    a_flat = _build_adjacency(adj_flat, vals).reshape(N * N // 128, 128)
    h_arr = jnp.asarray(h, jnp.float32).reshape(1)
    return _dense_cayley(h_arr, a_flat, x, W0, W_real, W_imag)


# speedup vs baseline: 6.1743x; 1.0064x over previous
import jax
import jax.numpy as jnp
from jax import lax
from jax.experimental import pallas as pl
from jax.experimental.pallas import tpu as pltpu
from jax.experimental.pallas import tpu_sc as plsc

N = 1024
E = 16384
FDIM = 128
JACOBI = 3

_NC, _NS = 2, 16
_EPT = E // _NS
_CHUNKS = _EPT // 128
_HALF = (N * N) // _NC
_WPT = _HALF // _NS
_ZBUF = 8192
_DUMP = _HALF


def _sc_scatter_kernel(adjf_hbm, vals_hbm, a_hbm,
                       rows_v, cols_v, vals_v, idx_v, zb_v, acc_sh,
                       sem_rc, sem_v, sem_z):
    cid = lax.axis_index("c")
    sid = lax.axis_index("s")
    base = sid * _EPT

    c_r = pltpu.async_copy(adjf_hbm.at[pl.ds(base, _EPT)], rows_v, sem_rc)
    c_c = pltpu.async_copy(adjf_hbm.at[pl.ds(E + base, _EPT)], cols_v, sem_rc)
    c_v = pltpu.async_copy(vals_hbm.at[pl.ds(base, _EPT)], vals_v, sem_v)

    zeros16 = jnp.zeros((16,), jnp.float32)

    def _zfill(i, carry):
        zb_v[pl.ds(i * 16, 16)] = zeros16
        return carry

    lax.fori_loop(0, _ZBUF // 16, _zfill, 0)
    zcs = [pltpu.async_copy(
        zb_v, acc_sh.at[pl.ds(sid * _WPT + j * _ZBUF, _ZBUF)], sem_z)
        for j in range(_WPT // _ZBUF)]

    c_r.wait()
    c_c.wait()
    row_lo = cid * (N // _NC)
    for j in range(_CHUNKS):
        for k in range(128 // 16):
            e = j * 128 + k * 16
            r = rows_v[pl.ds(e, 16)] - row_lo
            c = cols_v[pl.ds(e, 16)]
            flat = r * N + c
            ok = (r >= 0) & (r < (N // _NC))
            idx_v[j, pl.ds(k * 16, 16)] = jnp.where(ok, flat, _DUMP + sid * 16)

    c_v.wait()
    for z in zcs:
        z.wait()

    plsc.subcore_barrier()

    scs = [pltpu.async_copy(vals_v.at[pl.ds(j * 128, 128)],
                            acc_sh.at[idx_v.at[j]], sem_v, add=True)
           for j in range(_CHUNKS)]
    for s in scs:
        s.wait()

    plsc.subcore_barrier()

    pltpu.sync_copy(acc_sh.at[pl.ds(sid * _WPT, _WPT)],
                    a_hbm.at[pl.ds(cid * _HALF + sid * _WPT, _WPT)])


def _build_adjacency(adj_flat, vals):
    mesh = plsc.VectorSubcoreMesh(core_axis_name="c", subcore_axis_name="s")
    fn = pl.kernel(
        _sc_scatter_kernel,
        mesh=mesh,
        out_type=jax.ShapeDtypeStruct((N * N,), jnp.float32),
        scratch_types=[
            pltpu.VMEM((_EPT,), jnp.int32),
            pltpu.VMEM((_EPT,), jnp.int32),
            pltpu.VMEM((_EPT,), jnp.float32),
            pltpu.VMEM((_CHUNKS, 128), jnp.int32),
            pltpu.VMEM((_ZBUF,), jnp.float32),
            pltpu.VMEM_SHARED((_HALF + 16 * _NS,), jnp.float32),
            pltpu.SemaphoreType.DMA,
            pltpu.SemaphoreType.DMA,
            pltpu.SemaphoreType.DMA,
        ],
    )
    return fn(adj_flat, vals)


def _tc_kernel(h_ref, a_hbm, x_ref, w0_ref, wr_ref, wi_ref, o_ref,
               a_vmem, a_sem):
    h = h_ref[0]
    x = x_ref[:]

    cp = pltpu.make_async_copy(a_hbm.reshape(N, N), a_vmem, a_sem)
    cp.start()

    wcat = jnp.concatenate(
        [w0_ref[:], wr_ref[1], wi_ref[1], wr_ref[0], wi_ref[0]], axis=1)
    p = jnp.dot(x, wcat, preferred_element_type=jnp.float32)

    cp.wait()
    A = a_vmem[:]

    dr = h * (1.0 - jnp.sum(A, axis=1))
    denom = dr * dr + 1.0
    ar = (dr / denom)[:, None]
    ai = (-1.0 / denom)[:, None]

    def S(u):
        return h * (u - jnp.dot(A, u, preferred_element_type=jnp.float32))

    def m_apply(v):
        sv = S(v)
        vr, vi = v[:, :FDIM], v[:, FDIM:]
        xr = sv[:, :FDIM] + vi
        xi = sv[:, FDIM:] - vr
        yr, yi = xr, xi
        for _ in range(JACOBI):
            t = S(jnp.concatenate([yr, yi], axis=1))
            rr = xr - (t[:, :FDIM] - yi)
            ri = xi - (t[:, FDIM:] + yr)
            yr = yr + ar * rr - ai * ri
            yi = yi + ar * ri + ai * rr
        return yr, yi

    p0 = p[:, :FDIM]
    zr, zi = m_apply(p[:, FDIM:3 * FDIM])
    yr, _ = m_apply(
        jnp.concatenate([p[:, 3 * FDIM:4 * FDIM] + zr,
                         p[:, 4 * FDIM:] + zi], axis=1))
    o_ref[:] = jnp.maximum(p0 + 2.0 * yr, 0.0)


def _dense_cayley(h, a_flat, x, w0, wr, wi):
    vspec = pl.BlockSpec(memory_space=pltpu.VMEM)
    return pl.pallas_call(
        _tc_kernel,
        out_shape=jax.ShapeDtypeStruct((N, FDIM), jnp.float32),
        in_specs=[pl.BlockSpec(memory_space=pltpu.SMEM),
                  pl.BlockSpec(memory_space=pl.ANY),
                  vspec, vspec, vspec, vspec],
        out_specs=vspec,
        scratch_shapes=[pltpu.VMEM((N, N), jnp.float32),
                        pltpu.SemaphoreType.DMA],
    )(h, a_flat, x, w0, wr, wi)


def kernel(x, adj_indices, adj_values, h, W0, W_real, W_imag):
    adj_flat = adj_indices.astype(jnp.int32).reshape(2 * E)
    vals = adj_values.astype(jnp.float32)
    a_flat = _build_adjacency(adj_flat, vals).reshape(N * N // 128, 128)
    h_arr = jnp.asarray(h, jnp.float32).reshape(1)
    return _dense_cayley(h_arr, a_flat, x, W0, W_real, W_imag)
